# baseline (speedup 1.0000x reference)
import functools

import jax
import jax.numpy as jnp
from jax import lax
from jax.experimental import pallas as pl
from jax.experimental.pallas import tpu as pltpu
from jax.experimental.pallas import tpu_sc as plsc

_B = 4096
_L = 200
_D = 64
_DP = 128
_NW = 32
_BPW = _B // _NW
_NSLOT = 3
_LEAD = 2


def _make_sc_call():
  mesh = plsc.VectorSubcoreMesh(core_axis_name="c", subcore_axis_name="s")

  @functools.partial(
      pl.kernel,
      out_type=jax.ShapeDtypeStruct((_L * _D, _B), jnp.float32),
      mesh=mesh,
      compiler_params=pltpu.CompilerParams(use_tc_tiling_on_sc=True,
                                           needs_layout_passes=False),
      scratch_types=[
          pltpu.VMEM((_L, _BPW), jnp.int32),
          pltpu.VMEM((_L, _D), jnp.float32),
      ] + [pltpu.VMEM((_BPW, _DP), jnp.float32) for _ in range(_NSLOT)]
        + [pltpu.VMEM((_D, _BPW), jnp.float32) for _ in range(_NSLOT)]
        + [pltpu.SemaphoreType.DMA for _ in range(2 * _NSLOT)],
  )
  def sc_embed(xt_hbm, tok_hbm, pos_hbm, out_hbm, idx_v, pos_v, *bufs_sems):
    rows = bufs_sems[:_NSLOT]
    tbuf = bufs_sems[_NSLOT:2 * _NSLOT]
    gsem = bufs_sems[2 * _NSLOT:3 * _NSLOT]
    wsem = bufs_sems[3 * _NSLOT:]
    nc = 2
    wid = lax.axis_index("s") * nc + lax.axis_index("c")
    b_base = wid * _BPW

    pltpu.sync_copy(xt_hbm.at[:, pl.ds(b_base, _BPW)], idx_v)
    pltpu.sync_copy(pos_hbm.at[pl.ds(0, _L)], pos_v)

    lane = lax.iota(jnp.int32, 16)

    def process(l, rows_s, tb):
      lsplat = jnp.full((16,), l, jnp.int32)

      def d_body(d, _):
        dcol = jnp.full((16,), d, jnp.int32)
        pv = plsc.load_gather(pos_v, [lsplat, dcol])
        for g in range(_BPW // 16):
          v = plsc.load_gather(rows_s, [lane + g * 16, dcol])
          tb[d, pl.ds(g * 16, 16)] = v + pv
        return 0

      lax.fori_loop(0, _D, d_body, 0, unroll=2)

    def gather(c, s):
      pltpu.async_copy(tok_hbm.at[idx_v.at[c]], rows[s], gsem[s])

    def gwait(c, s):
      pltpu.make_async_copy(tok_hbm.at[idx_v.at[c]], rows[s], gsem[s]).wait()

    def wstart(c, s):
      pltpu.async_copy(
          tbuf[s], out_hbm.at[pl.ds(c * _D, _D), pl.ds(b_base, _BPW)],
          wsem[s])

    def wwait(c, s):
      pltpu.make_async_copy(
          tbuf[s], out_hbm.at[pl.ds(c * _D, _D), pl.ds(b_base, _BPW)],
          wsem[s]).wait()

    for s in range(_LEAD):
      gather(s, s)


    def group_body(g, _):
      for s0 in range(_NSLOT):
        c = g * _NSLOT + s0
        sl = (s0 + _LEAD) % _NSLOT
        cl = c + _LEAD

        @pl.when(cl < _L)
        def _():
          gather(cl, sl)

        gwait(c, s0)

        @pl.when(c >= _NSLOT)
        def _():
          wwait(c - _NSLOT, s0)

        process(c, rows[s0], tbuf[s0])
        wstart(c, s0)
      return 0

    lax.fori_loop(0, _L // _NSLOT, group_body, 0)
    for t in range(_L - (_L // _NSLOT) * _NSLOT):
      c = (_L // _NSLOT) * _NSLOT + t
      s0 = c % _NSLOT
      gwait(c, s0)
      wwait(c - _NSLOT, s0)
      process(c, rows[s0], tbuf[s0])
      wstart(c, s0)

    for t in range(_NSLOT):
      c = _L - _NSLOT + t
      wwait(c, c % _NSLOT)

  return sc_embed


_sc_embed = _make_sc_call()


@jax.jit
def kernel(x, tok_emb, pos_emb):
  xt = x.astype(jnp.int32).T
  tok128 = jnp.pad(tok_emb, ((0, 0), (0, _DP - _D)))
  out_t = _sc_embed(xt, tok128, pos_emb)
  return out_t.reshape(_L, _D, _B).transpose(2, 0, 1)


# speedup vs baseline: 1.7386x; 1.7386x over previous
import functools

import jax
import jax.numpy as jnp
from jax import lax
from jax.experimental import pallas as pl
from jax.experimental.pallas import tpu as pltpu
from jax.experimental.pallas import tpu_sc as plsc

_B = 4096
_L = 200
_D = 64
_DP = 128
_NW = 32
_BPW = _B // _NW
_NSLOT = 3
_LEAD = 2


def _make_sc_call():
  mesh = plsc.VectorSubcoreMesh(core_axis_name="c", subcore_axis_name="s")

  @functools.partial(
      pl.kernel,
      out_type=jax.ShapeDtypeStruct((_L * _D, _B), jnp.float32),
      mesh=mesh,
      compiler_params=pltpu.CompilerParams(use_tc_tiling_on_sc=True,
                                           needs_layout_passes=False),
      scratch_types=[
          pltpu.VMEM((_L, _BPW), jnp.int32),
          pltpu.VMEM((_L, _D), jnp.float32),
      ] + [pltpu.VMEM((_BPW, _DP), jnp.float32) for _ in range(_NSLOT)]
        + [pltpu.VMEM((_D, _BPW), jnp.float32) for _ in range(_NSLOT)]
        + [pltpu.SemaphoreType.DMA for _ in range(2 * _NSLOT)],
  )
  def sc_embed(xt_hbm, tok_hbm, pos_hbm, out_hbm, idx_v, pos_v, *bufs_sems):
    rows = bufs_sems[:_NSLOT]
    tbuf = bufs_sems[_NSLOT:2 * _NSLOT]
    gsem = bufs_sems[2 * _NSLOT:3 * _NSLOT]
    wsem = bufs_sems[3 * _NSLOT:]
    nc = 2
    wid = lax.axis_index("s") * nc + lax.axis_index("c")
    b_base = wid * _BPW

    pltpu.sync_copy(xt_hbm.at[:, pl.ds(b_base, _BPW)], idx_v)
    pltpu.sync_copy(pos_hbm.at[pl.ds(0, _L)], pos_v)

    lane = lax.iota(jnp.int32, 16)

    def process(l, rows_s, tb):
      lsplat = jnp.full((16,), l, jnp.int32)

      def pass_body(t, _):
        d0 = (t // 16) * 16
        j = t - (t // 16) * 16
        dcol = d0 + lax.rem(lane + j, 16)
        pv = plsc.load_gather(pos_v, [lsplat, dcol])
        for bb in range(_BPW // 16):
          brow = lane + bb * 16
          v = plsc.load_gather(rows_s, [brow, dcol])
          plsc.store_scatter(tb, [dcol, brow], v + pv)
        return 0

      lax.fori_loop(0, _D, pass_body, 0)

    def gather(c, s):
      pltpu.async_copy(tok_hbm.at[idx_v.at[c]], rows[s], gsem[s])

    def gwait(c, s):
      pltpu.make_async_copy(tok_hbm.at[idx_v.at[c]], rows[s], gsem[s]).wait()

    def wstart(c, s):
      pltpu.async_copy(
          tbuf[s], out_hbm.at[pl.ds(c * _D, _D), pl.ds(b_base, _BPW)],
          wsem[s])

    def wwait(c, s):
      pltpu.make_async_copy(
          tbuf[s], out_hbm.at[pl.ds(c * _D, _D), pl.ds(b_base, _BPW)],
          wsem[s]).wait()

    for s in range(_LEAD):
      gather(s, s)


    def group_body(g, _):
      for s0 in range(_NSLOT):
        c = g * _NSLOT + s0
        sl = (s0 + _LEAD) % _NSLOT
        cl = c + _LEAD

        @pl.when(cl < _L)
        def _():
          gather(cl, sl)

        gwait(c, s0)

        @pl.when(c >= _NSLOT)
        def _():
          wwait(c - _NSLOT, s0)

        process(c, rows[s0], tbuf[s0])
        wstart(c, s0)
      return 0

    lax.fori_loop(0, _L // _NSLOT, group_body, 0)
    for t in range(_L - (_L // _NSLOT) * _NSLOT):
      c = (_L // _NSLOT) * _NSLOT + t
      s0 = c % _NSLOT
      gwait(c, s0)
      wwait(c - _NSLOT, s0)
      process(c, rows[s0], tbuf[s0])
      wstart(c, s0)

    for t in range(_NSLOT):
      c = _L - _NSLOT + t
      wwait(c, c % _NSLOT)

  return sc_embed


_sc_embed = _make_sc_call()


@jax.jit
def kernel(x, tok_emb, pos_emb):
  xt = x.astype(jnp.int32).T
  tok128 = jnp.pad(tok_emb, ((0, 0), (0, _DP - _D)))
  out_t = _sc_embed(xt, tok128, pos_emb)
  return out_t.reshape(_L, _D, _B).transpose(2, 0, 1)


# speedup vs baseline: 1.7428x; 1.0024x over previous
import functools

import jax
import jax.numpy as jnp
from jax import lax
from jax.experimental import pallas as pl
from jax.experimental.pallas import tpu as pltpu
from jax.experimental.pallas import tpu_sc as plsc

_B = 4096
_L = 200
_D = 64
_DP = 128
_NW = 32
_BPW = _B // _NW
_NSLOT = 3
_LEAD = 2


def _make_sc_call():
  mesh = plsc.VectorSubcoreMesh(core_axis_name="c", subcore_axis_name="s")

  @functools.partial(
      pl.kernel,
      out_type=jax.ShapeDtypeStruct((_L * _D, _B), jnp.float32),
      mesh=mesh,
      compiler_params=pltpu.CompilerParams(use_tc_tiling_on_sc=True,
                                           needs_layout_passes=False),
      scratch_types=[
          pltpu.VMEM((_L, _BPW), jnp.int32),
          pltpu.VMEM((_L, _D), jnp.float32),
      ] + [pltpu.VMEM((_BPW, _DP), jnp.float32) for _ in range(_NSLOT)]
        + [pltpu.VMEM((_D, _BPW), jnp.float32) for _ in range(_NSLOT)]
        + [pltpu.SemaphoreType.DMA for _ in range(2 * _NSLOT)],
  )
  def sc_embed(xt_hbm, tok_hbm, pos_hbm, out_hbm, idx_v, pos_v, *bufs_sems):
    rows = bufs_sems[:_NSLOT]
    tbuf = bufs_sems[_NSLOT:2 * _NSLOT]
    gsem = bufs_sems[2 * _NSLOT:3 * _NSLOT]
    wsem = bufs_sems[3 * _NSLOT:]
    nc = 2
    wid = lax.axis_index("s") * nc + lax.axis_index("c")
    b_base = wid * _BPW

    pltpu.sync_copy(xt_hbm.at[:, pl.ds(b_base, _BPW)], idx_v)
    pltpu.sync_copy(pos_hbm.at[pl.ds(0, _L)], pos_v)

    lane = lax.iota(jnp.int32, 16)

    def process(l, rows_s, tb):
      lsplat = jnp.full((16,), l, jnp.int32)

      def pass_body(t, _):
        d0 = (t // 16) * 16
        j = t - (t // 16) * 16
        dcol = d0 + lax.rem(lane + j, 16)
        pv = plsc.load_gather(pos_v, [lsplat, dcol])
        for bb in range(_BPW // 16):
          brow = lane + bb * 16
          v = plsc.load_gather(rows_s, [brow, dcol])
          plsc.store_scatter(tb, [dcol, brow], v + pv)
        return 0

      lax.fori_loop(0, _D, pass_body, 0, unroll=2)

    def gather(c, s):
      pltpu.async_copy(tok_hbm.at[idx_v.at[c]], rows[s], gsem[s])

    def gwait(c, s):
      pltpu.make_async_copy(tok_hbm.at[idx_v.at[c]], rows[s], gsem[s]).wait()

    def wstart(c, s):
      pltpu.async_copy(
          tbuf[s], out_hbm.at[pl.ds(c * _D, _D), pl.ds(b_base, _BPW)],
          wsem[s])

    def wwait(c, s):
      pltpu.make_async_copy(
          tbuf[s], out_hbm.at[pl.ds(c * _D, _D), pl.ds(b_base, _BPW)],
          wsem[s]).wait()

    for s in range(_LEAD):
      gather(s, s)


    def group_body(g, _):
      for s0 in range(_NSLOT):
        c = g * _NSLOT + s0
        sl = (s0 + _LEAD) % _NSLOT
        cl = c + _LEAD

        @pl.when(cl < _L)
        def _():
          gather(cl, sl)

        gwait(c, s0)

        @pl.when(c >= _NSLOT)
        def _():
          wwait(c - _NSLOT, s0)

        process(c, rows[s0], tbuf[s0])
        wstart(c, s0)
      return 0

    lax.fori_loop(0, _L // _NSLOT, group_body, 0)
    for t in range(_L - (_L // _NSLOT) * _NSLOT):
      c = (_L // _NSLOT) * _NSLOT + t
      s0 = c % _NSLOT
      gwait(c, s0)
      wwait(c - _NSLOT, s0)
      process(c, rows[s0], tbuf[s0])
      wstart(c, s0)

    for t in range(_NSLOT):
      c = _L - _NSLOT + t
      wwait(c, c % _NSLOT)

  return sc_embed


_sc_embed = _make_sc_call()


@jax.jit
def kernel(x, tok_emb, pos_emb):
  xt = x.astype(jnp.int32).T
  tok128 = jnp.pad(tok_emb, ((0, 0), (0, _DP - _D)))
  out_t = _sc_embed(xt, tok128, pos_emb)
  return out_t.reshape(_L, _D, _B).transpose(2, 0, 1)


# speedup vs baseline: 2.4650x; 1.4144x over previous
import functools

import jax
import jax.numpy as jnp
from jax import lax
from jax.experimental import pallas as pl
from jax.experimental.pallas import tpu as pltpu
from jax.experimental.pallas import tpu_sc as plsc

_B = 4096
_L = 200
_D = 64
_DP = 128
_NW = 32
_BPW = _B // _NW
_NSLOT = 3
_LEAD = 2


def _make_sc_call():
  mesh = plsc.VectorSubcoreMesh(core_axis_name="c", subcore_axis_name="s")

  @functools.partial(
      pl.kernel,
      out_type=jax.ShapeDtypeStruct((_L * _D, _B), jnp.float32),
      mesh=mesh,
      compiler_params=pltpu.CompilerParams(use_tc_tiling_on_sc=True,
                                           needs_layout_passes=False),
      scratch_types=[
          pltpu.VMEM((_L, _BPW), jnp.int32),
          pltpu.VMEM((_L, _D), jnp.float32),
      ] + [pltpu.VMEM((_BPW, _DP), jnp.float32) for _ in range(_NSLOT)]
        + [pltpu.VMEM((_D, _BPW), jnp.float32) for _ in range(_NSLOT)]
        + [pltpu.SemaphoreType.DMA for _ in range(2 * _NSLOT)],
  )
  def sc_embed(xt_hbm, tok_hbm, pos_hbm, out_hbm, idx_v, pos_v, *bufs_sems):
    rows = bufs_sems[:_NSLOT]
    tbuf = bufs_sems[_NSLOT:2 * _NSLOT]
    gsem = bufs_sems[2 * _NSLOT:3 * _NSLOT]
    wsem = bufs_sems[3 * _NSLOT:]
    nc = 2
    wid = lax.axis_index("s") * nc + lax.axis_index("c")
    b_base = wid * _BPW

    pltpu.sync_copy(xt_hbm.at[:, pl.ds(b_base, _BPW)], idx_v)
    pltpu.sync_copy(pos_hbm.at[pl.ds(0, _L)], pos_v)

    lane = lax.iota(jnp.int32, 16)

    def process(l, rows_s, tb):
      lsplat = jnp.full((16,), l, jnp.int32)

      @plsc.parallel_loop(0, _D, 1, unroll=4)
      def pass_body(t):
        d0 = (t // 16) * 16
        j = t - (t // 16) * 16
        dcol = d0 + lax.rem(lane + j, 16)
        pv = plsc.load_gather(pos_v, [lsplat, dcol])
        for bb in range(_BPW // 16):
          brow = lane + bb * 16
          v = plsc.load_gather(rows_s, [brow, dcol])
          plsc.store_scatter(tb, [dcol, brow], v + pv)

    def gather(c, s):
      pltpu.async_copy(tok_hbm.at[idx_v.at[c]], rows[s], gsem[s])

    def gwait(c, s):
      pltpu.make_async_copy(tok_hbm.at[idx_v.at[c]], rows[s], gsem[s]).wait()

    def wstart(c, s):
      pltpu.async_copy(
          tbuf[s], out_hbm.at[pl.ds(c * _D, _D), pl.ds(b_base, _BPW)],
          wsem[s])

    def wwait(c, s):
      pltpu.make_async_copy(
          tbuf[s], out_hbm.at[pl.ds(c * _D, _D), pl.ds(b_base, _BPW)],
          wsem[s]).wait()

    for s in range(_LEAD):
      gather(s, s)


    def group_body(g, _):
      for s0 in range(_NSLOT):
        c = g * _NSLOT + s0
        sl = (s0 + _LEAD) % _NSLOT
        cl = c + _LEAD

        @pl.when(cl < _L)
        def _():
          gather(cl, sl)

        gwait(c, s0)

        @pl.when(c >= _NSLOT)
        def _():
          wwait(c - _NSLOT, s0)

        process(c, rows[s0], tbuf[s0])
        wstart(c, s0)
      return 0

    lax.fori_loop(0, _L // _NSLOT, group_body, 0)
    for t in range(_L - (_L // _NSLOT) * _NSLOT):
      c = (_L // _NSLOT) * _NSLOT + t
      s0 = c % _NSLOT
      gwait(c, s0)
      wwait(c - _NSLOT, s0)
      process(c, rows[s0], tbuf[s0])
      wstart(c, s0)

    for t in range(_NSLOT):
      c = _L - _NSLOT + t
      wwait(c, c % _NSLOT)

  return sc_embed


_sc_embed = _make_sc_call()


@jax.jit
def kernel(x, tok_emb, pos_emb):
  xt = x.astype(jnp.int32).T
  tok128 = jnp.pad(tok_emb, ((0, 0), (0, _DP - _D)))
  out_t = _sc_embed(xt, tok128, pos_emb)
  return out_t.reshape(_L, _D, _B).transpose(2, 0, 1)
